# baseline (speedup 1.0000x reference)
import functools

import jax
import jax.numpy as jnp
from jax.experimental import pallas as pl
from jax.experimental.pallas import tpu as pltpu

TL = 0.5
TH = 2.0
MIN_E = 1
BLK = 512
BLKR = 512

INTERPRET = False


def _router_body(bd2_ref, x_ref, wd1_ref, bd1_ref, wd2_ref, wg_ref, bg_ref,
                 w_ref, mask_ref, pos_ref, carry_ref):
    E = w_ref.shape[1]

    @pl.when(pl.program_id(0) == 0)
    def _():
        carry_ref[...] = jnp.zeros_like(carry_ref)

    xb = x_ref[...]
    h = jnp.dot(xb, wd1_ref[...], preferred_element_type=jnp.float32)
    h = jnp.maximum(h + bd1_ref[...], 0.0)
    zent = jnp.dot(h, wd2_ref[...],
                   preferred_element_type=jnp.float32)[:, :1] + bd2_ref[0]
    ent = jnp.maximum(zent, 0.0) + jnp.log1p(jnp.exp(-jnp.abs(zent)))
    ne = jnp.clip((ent - TL) / (TH - TL), 0.0, 1.0)
    kk = jnp.clip(jnp.round(MIN_E + ne * (E - MIN_E)), float(MIN_E), float(E))

    logits = jnp.dot(xb, wg_ref[...], preferred_element_type=jnp.float32)
    logits = logits + bg_ref[...]
    lane = jax.lax.broadcasted_iota(jnp.int32, logits.shape, 1)
    rank = jnp.zeros_like(logits)
    for ep in range(E):
        lep = logits[:, ep:ep + 1]
        rank += (lep > logits).astype(jnp.float32)
        rank += jnp.logical_and(lep == logits, ep < lane).astype(jnp.float32)
    mask = rank < kk
    z = jnp.where(mask, logits, -1e30)
    zmax = jnp.max(z, axis=1, keepdims=True)
    p = jnp.exp(z - zmax)
    w = jnp.where(mask, p / jnp.sum(p, axis=1, keepdims=True), 0.0)

    maskf = mask.astype(jnp.float32)
    n = xb.shape[0]
    r_iota = jax.lax.broadcasted_iota(jnp.int32, (n, n), 0)
    c_iota = jax.lax.broadcasted_iota(jnp.int32, (n, n), 1)
    tri = (c_iota < r_iota).astype(jnp.float32)
    pos_excl = jnp.dot(tri, maskf, preferred_element_type=jnp.float32)
    pos = carry_ref[...] + pos_excl
    carry_ref[...] += jnp.sum(maskf, axis=0, keepdims=True)

    w_ref[...] = w
    mask_ref[...] = mask.astype(jnp.int32)
    pos_ref[...] = pos.astype(jnp.int32)


def _router(xf, Wd1, bd1, Wd2, bd2, Wg, bg):
    T, D = xf.shape
    D2 = Wd1.shape[1]
    E = Wg.shape[1]
    grid = (T // BLK,)
    call = pl.pallas_call(
        _router_body,
        grid_spec=pltpu.PrefetchScalarGridSpec(
            num_scalar_prefetch=1,
            grid=grid,
            in_specs=[
                pl.BlockSpec((BLK, D), lambda i, s: (i, 0)),
                pl.BlockSpec((D, D2), lambda i, s: (0, 0)),
                pl.BlockSpec((1, D2), lambda i, s: (0, 0)),
                pl.BlockSpec((D2, 128), lambda i, s: (0, 0)),
                pl.BlockSpec((D, E), lambda i, s: (0, 0)),
                pl.BlockSpec((1, E), lambda i, s: (0, 0)),
            ],
            out_specs=[
                pl.BlockSpec((BLK, E), lambda i, s: (i, 0)),
                pl.BlockSpec((BLK, E), lambda i, s: (i, 0)),
                pl.BlockSpec((BLK, E), lambda i, s: (i, 0)),
            ],
            scratch_shapes=[pltpu.VMEM((1, E), jnp.float32)],
        ),
        out_shape=[
            jax.ShapeDtypeStruct((T, E), jnp.float32),
            jax.ShapeDtypeStruct((T, E), jnp.int32),
            jax.ShapeDtypeStruct((T, E), jnp.int32),
        ],
        interpret=INTERPRET,
    )
    wd2p = jnp.pad(Wd2.reshape(D2, 1), ((0, 0), (0, 127)))
    return call(bd2, xf, Wd1, bd1.reshape(1, D2), wd2p, Wg, bg.reshape(1, E))


def _ffn_body(nb_ref, xs_ref, we1_ref, be1_ref, we2_ref, be2_ref, ys_ref):
    b = pl.program_id(1)

    @pl.when(b < nb_ref[pl.program_id(0)])
    def _():
        xb = xs_ref[...].astype(jnp.bfloat16)
        h = jnp.dot(xb, we1_ref[0], preferred_element_type=jnp.float32)
        h = jnp.maximum(h + be1_ref[0], 0.0).astype(jnp.bfloat16)
        o = jnp.dot(h, we2_ref[0], preferred_element_type=jnp.float32)
        ys_ref[...] = o + be2_ref[0]


def _ffn(xs, We1b, be1, We2b, be2, nb, T):
    E, D, F = We1b.shape
    nbmax = T // BLKR

    def xs_map(e, b, nb):
        blk = jnp.maximum(jnp.minimum(b, nb[e] - 1), 0)
        return (e * nbmax + blk, 0)

    return pl.pallas_call(
        _ffn_body,
        grid_spec=pltpu.PrefetchScalarGridSpec(
            num_scalar_prefetch=1,
            grid=(E, nbmax),
            in_specs=[
                pl.BlockSpec((BLKR, D), xs_map),
                pl.BlockSpec((1, D, F), lambda e, b, nb: (e, 0, 0)),
                pl.BlockSpec((1, 1, F), lambda e, b, nb: (e, 0, 0)),
                pl.BlockSpec((1, F, D), lambda e, b, nb: (e, 0, 0)),
                pl.BlockSpec((1, 1, D), lambda e, b, nb: (e, 0, 0)),
            ],
            out_specs=pl.BlockSpec((BLKR, D), xs_map),
        ),
        out_shape=jax.ShapeDtypeStruct((E * T, D), jnp.float32),
        interpret=INTERPRET,
    )(nb, xs, We1b, be1.reshape(E, 1, F), We2b, be2.reshape(E, 1, D))


def kernel(x, Wd1, bd1, Wd2, bd2, Wg, bg, We1, be1, We2, be2):
    B, N, D = x.shape
    T = B * N
    E = Wg.shape[1]
    xf = x.reshape(T, D)

    w, mask, pos = _router(xf, Wd1, bd1, Wd2, bd2, Wg, bg)

    counts = pos[-1] + mask[-1]
    nb = (counts + BLKR - 1) // BLKR

    eidx = jnp.arange(E, dtype=jnp.int32)[None, :]
    dst = jnp.where(mask > 0, eidx * T + pos, E * T)
    src = jnp.broadcast_to(jnp.arange(T, dtype=jnp.int32)[:, None], (T, E))
    ids = jnp.zeros((E * T + 1,), jnp.int32).at[dst.reshape(-1)].set(
        src.reshape(-1), mode="drop")[:E * T]
    xs = xf[ids]

    ys = _ffn(xs, We1.astype(jnp.bfloat16), be1,
              We2.astype(jnp.bfloat16), be2, nb, T)

    slot = eidx * T + jnp.minimum(pos, T - 1)
    ysl = ys[slot.reshape(-1)].reshape(T, E, D)
    contrib = jnp.where((mask > 0)[..., None], ysl * w[..., None], 0.0)
    out = jnp.sum(contrib, axis=1)
    return out.reshape(B, N, D)
